# baseline (speedup 1.0000x reference)
import functools
import math

import jax
import jax.numpy as jnp
from jax import lax
from jax.experimental import pallas as pl
from jax.experimental.pallas import tpu as pltpu
from jax.experimental.pallas import tpu_sc as plsc

_BN_SCALE = 1.0 / math.sqrt(1.0 + 1e-5)

_N = 50000
_E = 800000
_BLK = 1000
_GRID = _N // _BLK
_NT = 16
_NS = 50048
_RPT = _NS // _NT
_EPAD = 819200
_TRASH = _N


def _full(spec_shape):
    nd = len(spec_shape)
    return pl.BlockSpec(spec_shape, lambda i, _n=nd: (0,) * _n)


def _rows(w):
    return pl.BlockSpec((_BLK, w), lambda i: (i, 0))


def _pre_body(feat, ctx, w1t, b1, w2t, b2, awt, ab, imp, lng, lnb,
              e1t, eb1, e2t, eb2, wlt, wrt, sb, pa_ref, pb_ref, r_ref):
    ce = jax.nn.relu(jnp.dot(ctx[...], w1t[...], preferred_element_type=jnp.float32) + b1[...])
    ce = jnp.dot(ce, w2t[...], preferred_element_type=jnp.float32) + b2[...]
    logits = jnp.dot(ce, awt[...], preferred_element_type=jnp.float32) + ab[...]
    att = jax.nn.softmax(logits, axis=-1)
    x = feat[...] * (att * imp[...])
    m = jnp.mean(x, axis=-1, keepdims=True)
    v = jnp.mean((x - m) * (x - m), axis=-1, keepdims=True)
    x = (x - m) / jnp.sqrt(v + 1e-5) * lng[...] + lnb[...]
    x = jax.nn.relu(jnp.dot(x, e1t[...], preferred_element_type=jnp.float32) + eb1[...])
    x = jax.nn.relu(jnp.dot(x, e2t[...], preferred_element_type=jnp.float32) + eb2[...])
    pp = jnp.dot(x, wlt[...], preferred_element_type=jnp.float32)
    hw = pp.shape[1] // 2
    pa_ref[...] = pp[:, :hw]
    pb_ref[...] = pp[:, hw:]
    r_ref[...] = jnp.dot(x, wrt[...], preferred_element_type=jnp.float32) + sb[...]


def _mid1_body(sa, sb_, r, c0, c1, bng, bnb, wlt, wrt, b_, pa_ref, pb_ref, r_ref, inv_ref):
    cnt = c0[...][:, 0:1] + c1[...][:, 0:1]
    inv = 1.0 / jnp.maximum(cnt, 1.0)
    s = jnp.concatenate([sa[...], sb_[...]], axis=1)
    x = jax.nn.relu((s * inv + r[...]) * (bng[...] * _BN_SCALE) + bnb[...])
    pp = jnp.dot(x, wlt[...], preferred_element_type=jnp.float32)
    hw = pp.shape[1] // 2
    pa_ref[...] = pp[:, :hw]
    pb_ref[...] = pp[:, hw:]
    r_ref[...] = jnp.dot(x, wrt[...], preferred_element_type=jnp.float32) + b_[...]
    inv_ref[...] = inv


def _mid2_body(sa, sb_, r, inv, bng, bnb, wlt, wrt, b_, pa_ref, pb_ref, r_ref):
    s = jnp.concatenate([sa[...], sb_[...]], axis=1)
    x = jax.nn.relu((s * inv[...] + r[...]) * (bng[...] * _BN_SCALE) + bnb[...])
    pp = jnp.dot(x, wlt[...], preferred_element_type=jnp.float32)
    hw = pp.shape[1] // 2
    pa_ref[...] = pp[:, :hw]
    pb_ref[...] = pp[:, hw:]
    r_ref[...] = jnp.dot(x, wrt[...], preferred_element_type=jnp.float32) + b_[...]


def _final_body(sa, sb_, r, inv, bng, bnb, w1t, b1, w2t, b2, out_ref):
    s = jnp.concatenate([sa[...], sb_[...]], axis=1)
    x = jax.nn.relu((s * inv[...] + r[...]) * (bng[...] * _BN_SCALE) + bnb[...])
    h = jax.nn.relu(jnp.dot(x, w1t[...], preferred_element_type=jnp.float32) + b1[...])
    z = jnp.dot(h, w2t[...], preferred_element_type=jnp.float32) + b2[...]
    out_ref[...] = jax.nn.sigmoid(z)


def _make_agg(wh, ch, m_per_slab):
    mesh = plsc.VectorSubcoreMesh(core_axis_name="c", subcore_axis_name="s")
    ept = _E // _NT
    n_chunks = ept // ch
    n_slab = n_chunks // m_per_slab
    M = m_per_slab

    @functools.partial(
        pl.kernel,
        out_type=[jax.ShapeDtypeStruct((_NS, wh), jnp.float32),
                  jax.ShapeDtypeStruct((_NS, wh), jnp.float32)],
        mesh=mesh,
        scratch_types=[
            pltpu.VMEM((M, ch), jnp.int32),
            pltpu.VMEM((M, ch), jnp.int32),
            pltpu.VMEM((ch, wh), jnp.float32),
            pltpu.VMEM((ch, wh), jnp.float32),
            pltpu.VMEM_SHARED((_NS, wh), jnp.float32),
            pltpu.SemaphoreType.DMA,
            pltpu.SemaphoreType.DMA,
            pltpu.SemaphoreType.DMA,
            pltpu.SemaphoreType.DMA,
        ],
        compiler_params=pltpu.CompilerParams(use_tc_tiling_on_sc=False),
    )
    def agg(pa_hbm, pb_hbm, src, dst, zeros_hbm, out0, out1,
            src2, dst2, rows0, rows1, acc, g0, g1, s0, s1):
        c = lax.axis_index("c")
        s = lax.axis_index("s")
        pltpu.sync_copy(zeros_hbm, acc.at[pl.ds(s * _RPT, _RPT)])
        plsc.subcore_barrier()
        r0 = s * n_chunks
        rbuf = (rows0, rows1)
        gsem = (g0, g1)
        ssem = (s0, s1)

        def pipeline(p_hbm):
            h_g = [None] * M
            h_s = [None] * M
            h_g[0] = pltpu.async_copy(p_hbm.at[src2.at[0]], rbuf[0], gsem[0])
            for m in range(M):
                h_g[m].wait()
                if m + 1 < M:
                    if m >= 1:
                        h_s[m - 1].wait()
                    h_g[m + 1] = pltpu.async_copy(
                        p_hbm.at[src2.at[m + 1]], rbuf[(m + 1) % 2], gsem[(m + 1) % 2])
                h_s[m] = pltpu.async_copy(
                    rbuf[m % 2], acc.at[dst2.at[m]], ssem[m % 2], add=True)
            if M >= 2:
                h_s[M - 2].wait()
            h_s[M - 1].wait()

        def slab(t, carry):
            row = r0 + t * M
            pltpu.sync_copy(src.at[pl.ds(row, M)], src2)
            pltpu.sync_copy(dst.at[pl.ds(row, M)], dst2)

            @pl.when(c == 0)
            def _():
                pipeline(pa_hbm)

            @pl.when(c == 1)
            def _():
                pipeline(pb_hbm)

            return carry

        lax.fori_loop(0, n_slab, slab, 0)
        plsc.subcore_barrier()

        @pl.when(c == 0)
        def _():
            pltpu.sync_copy(acc.at[pl.ds(s * _RPT, _RPT)],
                            out0.at[pl.ds(s * _RPT, _RPT)])

        @pl.when(c == 1)
        def _():
            pltpu.sync_copy(acc.at[pl.ds(s * _RPT, _RPT)],
                            out1.at[pl.ds(s * _RPT, _RPT)])

    return agg


def _make_cnt(ch):
    mesh = plsc.VectorSubcoreMesh(core_axis_name="c", subcore_axis_name="s")
    ept = _E // 2 // _NT
    n_chunks = ept // ch

    @functools.partial(
        pl.kernel,
        out_type=[jax.ShapeDtypeStruct((_NS, 16), jnp.float32),
                  jax.ShapeDtypeStruct((_NS, 16), jnp.float32)],
        mesh=mesh,
        scratch_types=[
            pltpu.VMEM((n_chunks, ch), jnp.int32),
            pltpu.VMEM((ch, 16), jnp.float32),
            pltpu.VMEM_SHARED((_NS, 16), jnp.float32),
            pltpu.SemaphoreType.DMA,
        ],
        compiler_params=pltpu.CompilerParams(use_tc_tiling_on_sc=False),
    )
    def cnt_k(dst, ones_hbm, zeros_hbm, out0, out1, dst2, ones_v, acc, sem):
        c = lax.axis_index("c")
        s = lax.axis_index("s")
        pltpu.sync_copy(ones_hbm, ones_v)
        pltpu.sync_copy(zeros_hbm, acc.at[pl.ds(s * _RPT, _RPT)])
        plsc.subcore_barrier()
        r0 = (c * (_E // 2) + s * ept) // ch
        pltpu.sync_copy(dst.at[pl.ds(r0, n_chunks)], dst2)
        hs = [pltpu.async_copy(ones_v, acc.at[dst2.at[m]], sem, add=True)
              for m in range(n_chunks)]
        for h in hs:
            h.wait()
        plsc.subcore_barrier()

        @pl.when(c == 0)
        def _():
            pltpu.sync_copy(acc.at[pl.ds(s * _RPT, _RPT)],
                            out0.at[pl.ds(s * _RPT, _RPT)])

        @pl.when(c == 1)
        def _():
            pltpu.sync_copy(acc.at[pl.ds(s * _RPT, _RPT)],
                            out1.at[pl.ds(s * _RPT, _RPT)])

    return cnt_k


def kernel(accessibility_features, edge_index, context_features, ctx_W1, ctx_b1,
           ctx_W2, ctx_b2, att_W, att_b, base_importance, ln_g, ln_b,
           enc_W1, enc_b1, enc_W2, enc_b2, sage1_Wl, sage1_Wr, sage1_b,
           bn1_g, bn1_b, sage2_Wl, sage2_Wr, sage2_b, bn2_g, bn2_b,
           sage3_Wl, sage3_Wr, sage3_b, bn3_g, bn3_b,
           svi_W1, svi_b1, svi_W2, svi_b2):
    f32 = jnp.float32
    r1c = lambda a: a.reshape(1, -1).astype(f32)

    ctx = jnp.pad(context_features, ((0, 0), (0, 3)))
    w1t = jnp.pad(ctx_W1.T, ((0, 3), (0, 0)))
    src = edge_index[0]
    dst = edge_index[1]
    zeros32 = jnp.zeros((_RPT, 32), f32)
    zeros16 = jnp.zeros((_RPT, 16), f32)
    ones16 = jnp.ones((1000, 16), f32)

    p1a, p1b, r1 = pl.pallas_call(
        _pre_body,
        grid=(_GRID,),
        in_specs=[_rows(128), _rows(8), _full((8, 32)), _full((1, 32)),
                  _full((32, 32)), _full((1, 32)), _full((32, 128)), _full((1, 128)),
                  _full((1, 128)), _full((1, 128)), _full((1, 128)),
                  _full((128, 64)), _full((1, 64)), _full((64, 64)), _full((1, 64)),
                  _full((64, 64)), _full((64, 64)), _full((1, 64))],
        out_specs=[_rows(32), _rows(32), _rows(64)],
        out_shape=[jax.ShapeDtypeStruct((_N, 32), f32),
                   jax.ShapeDtypeStruct((_N, 32), f32),
                   jax.ShapeDtypeStruct((_N, 64), f32)],
    )(accessibility_features, ctx, w1t, r1c(ctx_b1), ctx_W2.T, r1c(ctx_b2),
      att_W.T, r1c(att_b), r1c(base_importance), r1c(ln_g), r1c(ln_b),
      enc_W1.T, r1c(enc_b1), enc_W2.T, r1c(enc_b2),
      sage1_Wl.T, sage1_Wr.T, r1c(sage1_b))

    d1000 = dst.reshape(_E // 1000, 1000)
    cnt2 = _make_cnt(1000)(d1000, ones16, zeros16)

    s400 = src.reshape(_E // 400, 400)
    d400 = dst.reshape(_E // 400, 400)
    s1000 = src.reshape(_E // 1000, 1000)
    agg64 = _make_agg(32, 400, 5)
    s1 = agg64(p1a, p1b, s400, d400, zeros32)

    p2a, p2b, r2, inv = pl.pallas_call(
        _mid1_body,
        grid=(_GRID,),
        in_specs=[_rows(32), _rows(32), _rows(64), _rows(16), _rows(16),
                  _full((1, 64)), _full((1, 64)),
                  _full((64, 64)), _full((64, 64)), _full((1, 64))],
        out_specs=[_rows(32), _rows(32), _rows(64), _rows(1)],
        out_shape=[jax.ShapeDtypeStruct((_N, 32), f32),
                   jax.ShapeDtypeStruct((_N, 32), f32),
                   jax.ShapeDtypeStruct((_N, 64), f32),
                   jax.ShapeDtypeStruct((_N, 1), f32)],
    )(s1[0], s1[1], r1, cnt2[0], cnt2[1], r1c(bn1_g), r1c(bn1_b),
      sage2_Wl.T, sage2_Wr.T, r1c(sage2_b))

    s2 = agg64(p2a, p2b, s400, d400, zeros32)

    p3a, p3b, r3 = pl.pallas_call(
        _mid2_body,
        grid=(_GRID,),
        in_specs=[_rows(32), _rows(32), _rows(64), _rows(1),
                  _full((1, 64)), _full((1, 64)),
                  _full((64, 32)), _full((64, 32)), _full((1, 32))],
        out_specs=[_rows(16), _rows(16), _rows(32)],
        out_shape=[jax.ShapeDtypeStruct((_N, 16), f32),
                   jax.ShapeDtypeStruct((_N, 16), f32),
                   jax.ShapeDtypeStruct((_N, 32), f32)],
    )(s2[0], s2[1], r2, inv, r1c(bn2_g), r1c(bn2_b),
      sage3_Wl.T, sage3_Wr.T, r1c(sage3_b))

    s3 = _make_agg(16, 1000, 10)(p3a, p3b, s1000, d1000, zeros16)

    svi = pl.pallas_call(
        _final_body,
        grid=(_GRID,),
        in_specs=[_rows(16), _rows(16), _rows(32), _rows(1),
                  _full((1, 32)), _full((1, 32)),
                  _full((32, 16)), _full((1, 16)), _full((16, 1)), _full((1, 1))],
        out_specs=[_rows(1)],
        out_shape=[jax.ShapeDtypeStruct((_N, 1), f32)],
    )(s3[0], s3[1], r3, inv, r1c(bn3_g), r1c(bn3_b),
      svi_W1.T, r1c(svi_b1), svi_W2.T, r1c(svi_b2))[0]

    return svi[:, 0]


# speedup vs baseline: 1.0616x; 1.0616x over previous
import functools
import math

import jax
import jax.numpy as jnp
from jax import lax
from jax.experimental import pallas as pl
from jax.experimental.pallas import tpu as pltpu
from jax.experimental.pallas import tpu_sc as plsc

_BN_SCALE = 1.0 / math.sqrt(1.0 + 1e-5)

_N = 50000
_E = 800000
_BLK = 2000
_GRID = _N // _BLK
_NT = 16
_NS = 50048
_RPT = _NS // _NT
_EPAD = 819200
_TRASH = _N


def _full(spec_shape):
    nd = len(spec_shape)
    return pl.BlockSpec(spec_shape, lambda i, _n=nd: (0,) * _n)


def _rows(w):
    return pl.BlockSpec((_BLK, w), lambda i: (i, 0))


def _pre_body(feat, ctx, w1t, b1, w2t, b2, awt, ab, imp, lng, lnb,
              e1t, eb1, e2t, eb2, wlt, wrt, sb, pa_ref, pb_ref, r_ref):
    ce = jax.nn.relu(jnp.dot(ctx[...], w1t[...], preferred_element_type=jnp.float32) + b1[...])
    ce = jnp.dot(ce, w2t[...], preferred_element_type=jnp.float32) + b2[...]
    logits = jnp.dot(ce, awt[...], preferred_element_type=jnp.float32) + ab[...]
    att = jax.nn.softmax(logits, axis=-1)
    x = feat[...] * (att * imp[...])
    m = jnp.mean(x, axis=-1, keepdims=True)
    v = jnp.mean((x - m) * (x - m), axis=-1, keepdims=True)
    x = (x - m) / jnp.sqrt(v + 1e-5) * lng[...] + lnb[...]
    x = jax.nn.relu(jnp.dot(x, e1t[...], preferred_element_type=jnp.float32) + eb1[...])
    x = jax.nn.relu(jnp.dot(x, e2t[...], preferred_element_type=jnp.float32) + eb2[...])
    pp = jnp.dot(x, wlt[...], preferred_element_type=jnp.float32)
    hw = pp.shape[1] // 2
    pa_ref[...] = pp[:, :hw]
    pb_ref[...] = pp[:, hw:]
    r_ref[...] = jnp.dot(x, wrt[...], preferred_element_type=jnp.float32) + sb[...]


def _mid1_body(sa, sb_, r, c0, c1, bng, bnb, wlt, wrt, b_, pa_ref, pb_ref, r_ref, inv_ref):
    cnt = c0[...][:, 0:1] + c1[...][:, 0:1]
    inv = 1.0 / jnp.maximum(cnt, 1.0)
    s = jnp.concatenate([sa[...], sb_[...]], axis=1)
    x = jax.nn.relu((s * inv + r[...]) * (bng[...] * _BN_SCALE) + bnb[...])
    pp = jnp.dot(x, wlt[...], preferred_element_type=jnp.float32)
    hw = pp.shape[1] // 2
    pa_ref[...] = pp[:, :hw]
    pb_ref[...] = pp[:, hw:]
    r_ref[...] = jnp.dot(x, wrt[...], preferred_element_type=jnp.float32) + b_[...]
    inv_ref[...] = inv


def _mid2_body(sa, sb_, r, inv, bng, bnb, wlt, wrt, b_, pa_ref, pb_ref, r_ref):
    s = jnp.concatenate([sa[...], sb_[...]], axis=1)
    x = jax.nn.relu((s * inv[...] + r[...]) * (bng[...] * _BN_SCALE) + bnb[...])
    pp = jnp.dot(x, wlt[...], preferred_element_type=jnp.float32)
    hw = pp.shape[1] // 2
    pa_ref[...] = pp[:, :hw]
    pb_ref[...] = pp[:, hw:]
    r_ref[...] = jnp.dot(x, wrt[...], preferred_element_type=jnp.float32) + b_[...]


def _final_body(sa, sb_, r, inv, bng, bnb, w1t, b1, w2t, b2, out_ref):
    s = jnp.concatenate([sa[...], sb_[...]], axis=1)
    x = jax.nn.relu((s * inv[...] + r[...]) * (bng[...] * _BN_SCALE) + bnb[...])
    h = jax.nn.relu(jnp.dot(x, w1t[...], preferred_element_type=jnp.float32) + b1[...])
    z = jnp.dot(h, w2t[...], preferred_element_type=jnp.float32) + b2[...]
    out_ref[...] = jax.nn.sigmoid(z)


def _make_agg(wh, ch, m_per_slab):
    mesh = plsc.VectorSubcoreMesh(core_axis_name="c", subcore_axis_name="s")
    ept = _E // _NT
    n_chunks = ept // ch
    n_slab = n_chunks // m_per_slab
    M = m_per_slab

    @functools.partial(
        pl.kernel,
        out_type=[jax.ShapeDtypeStruct((_NS, wh), jnp.float32),
                  jax.ShapeDtypeStruct((_NS, wh), jnp.float32)],
        mesh=mesh,
        scratch_types=[
            pltpu.VMEM((M * ch,), jnp.int32),
            pltpu.VMEM((M * ch,), jnp.int32),
            pltpu.VMEM((ch, wh), jnp.float32),
            pltpu.VMEM((ch, wh), jnp.float32),
            pltpu.VMEM_SHARED((_NS, wh), jnp.float32),
            pltpu.SemaphoreType.DMA,
            pltpu.SemaphoreType.DMA,
            pltpu.SemaphoreType.DMA,
            pltpu.SemaphoreType.DMA,
        ],
        compiler_params=pltpu.CompilerParams(use_tc_tiling_on_sc=False),
    )
    def agg(pa_hbm, pb_hbm, src, dst, zeros_hbm, out0, out1,
            src2, dst2, rows0, rows1, acc, g0, g1, s0, s1):
        c = lax.axis_index("c")
        s = lax.axis_index("s")
        pltpu.sync_copy(zeros_hbm, acc.at[pl.ds(s * _RPT, _RPT)])
        plsc.subcore_barrier()
        r0 = s * n_chunks
        rbuf = (rows0, rows1)
        gsem = (g0, g1)
        ssem = (s0, s1)

        def pipeline(p_hbm):
            h_g = [None] * M
            h_s = [None] * M
            h_g[0] = pltpu.async_copy(
                p_hbm.at[src2.at[pl.ds(0, ch)]], rbuf[0], gsem[0])
            for m in range(M):
                h_g[m].wait()
                if m + 1 < M:
                    if m >= 1:
                        h_s[m - 1].wait()
                    h_g[m + 1] = pltpu.async_copy(
                        p_hbm.at[src2.at[pl.ds((m + 1) * ch, ch)]],
                        rbuf[(m + 1) % 2], gsem[(m + 1) % 2])
                h_s[m] = pltpu.async_copy(
                    rbuf[m % 2], acc.at[dst2.at[pl.ds(m * ch, ch)]],
                    ssem[m % 2], add=True)
            if M >= 2:
                h_s[M - 2].wait()
            h_s[M - 1].wait()

        def slab(t, carry):
            e = (r0 + t * M) * ch
            pltpu.sync_copy(src.at[pl.ds(e, M * ch)], src2)
            pltpu.sync_copy(dst.at[pl.ds(e, M * ch)], dst2)

            @pl.when(c == 0)
            def _():
                pipeline(pa_hbm)

            @pl.when(c == 1)
            def _():
                pipeline(pb_hbm)

            return carry

        lax.fori_loop(0, n_slab, slab, 0)
        plsc.subcore_barrier()

        @pl.when(c == 0)
        def _():
            pltpu.sync_copy(acc.at[pl.ds(s * _RPT, _RPT)],
                            out0.at[pl.ds(s * _RPT, _RPT)])

        @pl.when(c == 1)
        def _():
            pltpu.sync_copy(acc.at[pl.ds(s * _RPT, _RPT)],
                            out1.at[pl.ds(s * _RPT, _RPT)])

    return agg


def _make_cnt(ch):
    mesh = plsc.VectorSubcoreMesh(core_axis_name="c", subcore_axis_name="s")
    ept = _E // 2 // _NT
    n_chunks = ept // ch

    @functools.partial(
        pl.kernel,
        out_type=[jax.ShapeDtypeStruct((_NS, 16), jnp.float32),
                  jax.ShapeDtypeStruct((_NS, 16), jnp.float32)],
        mesh=mesh,
        scratch_types=[
            pltpu.VMEM((n_chunks * ch,), jnp.int32),
            pltpu.VMEM((ch, 16), jnp.float32),
            pltpu.VMEM_SHARED((_NS, 16), jnp.float32),
            pltpu.SemaphoreType.DMA,
        ],
        compiler_params=pltpu.CompilerParams(use_tc_tiling_on_sc=False),
    )
    def cnt_k(dst, ones_hbm, zeros_hbm, out0, out1, dst2, ones_v, acc, sem):
        c = lax.axis_index("c")
        s = lax.axis_index("s")
        pltpu.sync_copy(ones_hbm, ones_v)
        pltpu.sync_copy(zeros_hbm, acc.at[pl.ds(s * _RPT, _RPT)])
        plsc.subcore_barrier()
        e0 = c * (_E // 2) + s * ept
        pltpu.sync_copy(dst.at[pl.ds(e0, n_chunks * ch)], dst2)
        hs = [pltpu.async_copy(ones_v, acc.at[dst2.at[pl.ds(m * ch, ch)]],
                               sem, add=True)
              for m in range(n_chunks)]
        for h in hs:
            h.wait()
        plsc.subcore_barrier()

        @pl.when(c == 0)
        def _():
            pltpu.sync_copy(acc.at[pl.ds(s * _RPT, _RPT)],
                            out0.at[pl.ds(s * _RPT, _RPT)])

        @pl.when(c == 1)
        def _():
            pltpu.sync_copy(acc.at[pl.ds(s * _RPT, _RPT)],
                            out1.at[pl.ds(s * _RPT, _RPT)])

    return cnt_k


def kernel(accessibility_features, edge_index, context_features, ctx_W1, ctx_b1,
           ctx_W2, ctx_b2, att_W, att_b, base_importance, ln_g, ln_b,
           enc_W1, enc_b1, enc_W2, enc_b2, sage1_Wl, sage1_Wr, sage1_b,
           bn1_g, bn1_b, sage2_Wl, sage2_Wr, sage2_b, bn2_g, bn2_b,
           sage3_Wl, sage3_Wr, sage3_b, bn3_g, bn3_b,
           svi_W1, svi_b1, svi_W2, svi_b2):
    f32 = jnp.float32
    r1c = lambda a: a.reshape(1, -1).astype(f32)

    ctx = jnp.pad(context_features, ((0, 0), (0, 3)))
    w1t = jnp.pad(ctx_W1.T, ((0, 3), (0, 0)))
    src = edge_index[0]
    dst = edge_index[1]
    zeros32 = jnp.zeros((_RPT, 32), f32)
    zeros16 = jnp.zeros((_RPT, 16), f32)
    ones16 = jnp.ones((1000, 16), f32)

    p1a, p1b, r1 = pl.pallas_call(
        _pre_body,
        grid=(_GRID,),
        in_specs=[_rows(128), _rows(8), _full((8, 32)), _full((1, 32)),
                  _full((32, 32)), _full((1, 32)), _full((32, 128)), _full((1, 128)),
                  _full((1, 128)), _full((1, 128)), _full((1, 128)),
                  _full((128, 64)), _full((1, 64)), _full((64, 64)), _full((1, 64)),
                  _full((64, 64)), _full((64, 64)), _full((1, 64))],
        out_specs=[_rows(32), _rows(32), _rows(64)],
        out_shape=[jax.ShapeDtypeStruct((_N, 32), f32),
                   jax.ShapeDtypeStruct((_N, 32), f32),
                   jax.ShapeDtypeStruct((_N, 64), f32)],
    )(accessibility_features, ctx, w1t, r1c(ctx_b1), ctx_W2.T, r1c(ctx_b2),
      att_W.T, r1c(att_b), r1c(base_importance), r1c(ln_g), r1c(ln_b),
      enc_W1.T, r1c(enc_b1), enc_W2.T, r1c(enc_b2),
      sage1_Wl.T, sage1_Wr.T, r1c(sage1_b))

    cnt2 = _make_cnt(1000)(dst, ones16, zeros16)

    agg64 = _make_agg(32, 400, 5)
    s1 = agg64(p1a, p1b, src, dst, zeros32)

    p2a, p2b, r2, inv = pl.pallas_call(
        _mid1_body,
        grid=(_GRID,),
        in_specs=[_rows(32), _rows(32), _rows(64), _rows(16), _rows(16),
                  _full((1, 64)), _full((1, 64)),
                  _full((64, 64)), _full((64, 64)), _full((1, 64))],
        out_specs=[_rows(32), _rows(32), _rows(64), _rows(1)],
        out_shape=[jax.ShapeDtypeStruct((_N, 32), f32),
                   jax.ShapeDtypeStruct((_N, 32), f32),
                   jax.ShapeDtypeStruct((_N, 64), f32),
                   jax.ShapeDtypeStruct((_N, 1), f32)],
    )(s1[0], s1[1], r1, cnt2[0], cnt2[1], r1c(bn1_g), r1c(bn1_b),
      sage2_Wl.T, sage2_Wr.T, r1c(sage2_b))

    s2 = agg64(p2a, p2b, src, dst, zeros32)

    p3a, p3b, r3 = pl.pallas_call(
        _mid2_body,
        grid=(_GRID,),
        in_specs=[_rows(32), _rows(32), _rows(64), _rows(1),
                  _full((1, 64)), _full((1, 64)),
                  _full((64, 32)), _full((64, 32)), _full((1, 32))],
        out_specs=[_rows(16), _rows(16), _rows(32)],
        out_shape=[jax.ShapeDtypeStruct((_N, 16), f32),
                   jax.ShapeDtypeStruct((_N, 16), f32),
                   jax.ShapeDtypeStruct((_N, 32), f32)],
    )(s2[0], s2[1], r2, inv, r1c(bn2_g), r1c(bn2_b),
      sage3_Wl.T, sage3_Wr.T, r1c(sage3_b))

    s3 = _make_agg(16, 1000, 10)(p3a, p3b, src, dst, zeros16)

    svi = pl.pallas_call(
        _final_body,
        grid=(_GRID,),
        in_specs=[_rows(16), _rows(16), _rows(32), _rows(1),
                  _full((1, 32)), _full((1, 32)),
                  _full((32, 16)), _full((1, 16)), _full((16, 1)), _full((1, 1))],
        out_specs=[_rows(1)],
        out_shape=[jax.ShapeDtypeStruct((_N, 1), f32)],
    )(s3[0], s3[1], r3, inv, r1c(bn3_g), r1c(bn3_b),
      svi_W1.T, r1c(svi_b1), svi_W2.T, r1c(svi_b2))[0]

    return svi[:, 0]


# speedup vs baseline: 1.1597x; 1.0924x over previous
import functools
import math

import jax
import jax.numpy as jnp
from jax import lax
from jax.experimental import pallas as pl
from jax.experimental.pallas import tpu as pltpu
from jax.experimental.pallas import tpu_sc as plsc

_BN_SCALE = 1.0 / math.sqrt(1.0 + 1e-5)

_N = 50000
_E = 800000
_BLK = 2000
_GRID = _N // _BLK
_NT = 16
_NS = 50048
_RPT = _NS // _NT
_EPAD = 819200
_TRASH = _N


def _full(spec_shape):
    nd = len(spec_shape)
    return pl.BlockSpec(spec_shape, lambda i, _n=nd: (0,) * _n)


def _rows(w):
    return pl.BlockSpec((_BLK, w), lambda i: (i, 0))


def _pre_body(feat, ctx, w1t, b1, w2t, b2, awt, ab, imp, lng, lnb,
              e1t, eb1, e2t, eb2, wlt, wrt, sb, pa_ref, pb_ref, r_ref):
    ce = jax.nn.relu(jnp.dot(ctx[...], w1t[...], preferred_element_type=jnp.float32) + b1[...])
    ce = jnp.dot(ce, w2t[...], preferred_element_type=jnp.float32) + b2[...]
    logits = jnp.dot(ce, awt[...], preferred_element_type=jnp.float32) + ab[...]
    att = jax.nn.softmax(logits, axis=-1)
    x = feat[...] * (att * imp[...])
    m = jnp.mean(x, axis=-1, keepdims=True)
    v = jnp.mean((x - m) * (x - m), axis=-1, keepdims=True)
    x = (x - m) / jnp.sqrt(v + 1e-5) * lng[...] + lnb[...]
    x = jax.nn.relu(jnp.dot(x, e1t[...], preferred_element_type=jnp.float32) + eb1[...])
    x = jax.nn.relu(jnp.dot(x, e2t[...], preferred_element_type=jnp.float32) + eb2[...])
    pp = jnp.dot(x, wlt[...], preferred_element_type=jnp.float32)
    hw = pp.shape[1] // 2
    pa_ref[...] = pp[:, :hw]
    pb_ref[...] = pp[:, hw:]
    r_ref[...] = jnp.dot(x, wrt[...], preferred_element_type=jnp.float32) + sb[...]


def _mid1_body(sa, r, c01, bng, bnb, wlt, wrt, b_, pa_ref, pb_ref, r_ref, inv_ref):
    cc = c01[...]
    cnt = cc[:, 0:1] + cc[:, 16:17]
    inv = 1.0 / jnp.maximum(cnt, 1.0)
    x = jax.nn.relu((sa[...][:, :64] * inv + r[...]) * (bng[...] * _BN_SCALE) + bnb[...])
    pp = jnp.dot(x, wlt[...], preferred_element_type=jnp.float32)
    hw = pp.shape[1] // 2
    pa_ref[...] = pp[:, :hw]
    pb_ref[...] = pp[:, hw:]
    r_ref[...] = jnp.dot(x, wrt[...], preferred_element_type=jnp.float32) + b_[...]
    inv_ref[...] = inv


def _mid2_body(sa, r, inv, bng, bnb, wlt, wrt, b_, pa_ref, pb_ref, r_ref):
    x = jax.nn.relu((sa[...][:, :64] * inv[...] + r[...]) * (bng[...] * _BN_SCALE) + bnb[...])
    pp = jnp.dot(x, wlt[...], preferred_element_type=jnp.float32)
    hw = pp.shape[1] // 2
    pa_ref[...] = pp[:, :hw]
    pb_ref[...] = pp[:, hw:]
    r_ref[...] = jnp.dot(x, wrt[...], preferred_element_type=jnp.float32) + b_[...]


def _final_body(sa, r, inv, bng, bnb, w1t, b1, w2t, b2, out_ref):
    x = jax.nn.relu((sa[...][:, :32] * inv[...] + r[...]) * (bng[...] * _BN_SCALE) + bnb[...])
    h = jax.nn.relu(jnp.dot(x, w1t[...], preferred_element_type=jnp.float32) + b1[...])
    z = jnp.dot(h, w2t[...], preferred_element_type=jnp.float32) + b2[...]
    out_ref[...] = jax.nn.sigmoid(z)


def _make_agg(wh, ch, m_per_slab):
    mesh = plsc.VectorSubcoreMesh(core_axis_name="c", subcore_axis_name="s")
    ept = _E // _NT
    n_chunks = ept // ch
    n_slab = n_chunks // m_per_slab
    M = m_per_slab

    @functools.partial(
        pl.kernel,
        out_type=jax.ShapeDtypeStruct((_NS, 128), jnp.float32),
        mesh=mesh,
        scratch_types=[
            pltpu.VMEM((M * ch,), jnp.int32),
            pltpu.VMEM((M * ch,), jnp.int32),
            pltpu.VMEM((ch, wh), jnp.float32),
            pltpu.VMEM((ch, wh), jnp.float32),
            pltpu.VMEM_SHARED((_NS, wh), jnp.float32),
            pltpu.SemaphoreType.DMA,
            pltpu.SemaphoreType.DMA,
            pltpu.SemaphoreType.DMA,
            pltpu.SemaphoreType.DMA,
        ],
        compiler_params=pltpu.CompilerParams(use_tc_tiling_on_sc=False),
    )
    def agg(pa_hbm, pb_hbm, src, dst, zeros_hbm, out,
            src2, dst2, rows0, rows1, acc, g0, g1, s0, s1):
        c = lax.axis_index("c")
        s = lax.axis_index("s")
        pltpu.sync_copy(zeros_hbm, acc.at[pl.ds(s * _RPT, _RPT)])
        plsc.subcore_barrier()
        r0 = s * n_chunks
        rbuf = (rows0, rows1)
        gsem = (g0, g1)
        ssem = (s0, s1)

        def pipeline(p_hbm):
            h_g = [None] * M
            h_s = [None] * M
            h_g[0] = pltpu.async_copy(
                p_hbm.at[src2.at[pl.ds(0, ch)]], rbuf[0], gsem[0])
            for m in range(M):
                h_g[m].wait()
                if m + 1 < M:
                    if m >= 1:
                        h_s[m - 1].wait()
                    h_g[m + 1] = pltpu.async_copy(
                        p_hbm.at[src2.at[pl.ds((m + 1) * ch, ch)]],
                        rbuf[(m + 1) % 2], gsem[(m + 1) % 2])
                h_s[m] = pltpu.async_copy(
                    rbuf[m % 2], acc.at[dst2.at[pl.ds(m * ch, ch)]],
                    ssem[m % 2], add=True)
            if M >= 2:
                h_s[M - 2].wait()
            h_s[M - 1].wait()

        def slab(t, carry):
            e = (r0 + t * M) * ch
            pltpu.sync_copy(src.at[pl.ds(e, M * ch)], src2)
            pltpu.sync_copy(dst.at[pl.ds(e, M * ch)], dst2)

            @pl.when(c == 0)
            def _():
                pipeline(pa_hbm)

            @pl.when(c == 1)
            def _():
                pipeline(pb_hbm)

            return carry

        lax.fori_loop(0, n_slab, slab, 0)
        plsc.subcore_barrier()

        @pl.when(c == 0)
        def _():
            pltpu.sync_copy(acc.at[pl.ds(s * _RPT, _RPT)],
                            out.at[pl.ds(s * _RPT, _RPT), pl.ds(0, wh)])

        @pl.when(c == 1)
        def _():
            pltpu.sync_copy(acc.at[pl.ds(s * _RPT, _RPT)],
                            out.at[pl.ds(s * _RPT, _RPT), pl.ds(wh, wh)])

    return agg


def _make_cnt(ch):
    mesh = plsc.VectorSubcoreMesh(core_axis_name="c", subcore_axis_name="s")
    ept = _E // 2 // _NT
    n_chunks = ept // ch

    @functools.partial(
        pl.kernel,
        out_type=jax.ShapeDtypeStruct((_NS, 128), jnp.float32),
        mesh=mesh,
        scratch_types=[
            pltpu.VMEM((n_chunks * ch,), jnp.int32),
            pltpu.VMEM((ch, 16), jnp.float32),
            pltpu.VMEM_SHARED((_NS, 16), jnp.float32),
            pltpu.SemaphoreType.DMA,
        ],
        compiler_params=pltpu.CompilerParams(use_tc_tiling_on_sc=False),
    )
    def cnt_k(dst, ones_hbm, zeros_hbm, out, dst2, ones_v, acc, sem):
        c = lax.axis_index("c")
        s = lax.axis_index("s")
        pltpu.sync_copy(ones_hbm, ones_v)
        pltpu.sync_copy(zeros_hbm, acc.at[pl.ds(s * _RPT, _RPT)])
        plsc.subcore_barrier()
        e0 = c * (_E // 2) + s * ept
        pltpu.sync_copy(dst.at[pl.ds(e0, n_chunks * ch)], dst2)
        hs = [pltpu.async_copy(ones_v, acc.at[dst2.at[pl.ds(m * ch, ch)]],
                               sem, add=True)
              for m in range(n_chunks)]
        for h in hs:
            h.wait()
        plsc.subcore_barrier()

        @pl.when(c == 0)
        def _():
            pltpu.sync_copy(acc.at[pl.ds(s * _RPT, _RPT)],
                            out.at[pl.ds(s * _RPT, _RPT), pl.ds(0, 16)])

        @pl.when(c == 1)
        def _():
            pltpu.sync_copy(acc.at[pl.ds(s * _RPT, _RPT)],
                            out.at[pl.ds(s * _RPT, _RPT), pl.ds(16, 16)])

    return cnt_k


def kernel(accessibility_features, edge_index, context_features, ctx_W1, ctx_b1,
           ctx_W2, ctx_b2, att_W, att_b, base_importance, ln_g, ln_b,
           enc_W1, enc_b1, enc_W2, enc_b2, sage1_Wl, sage1_Wr, sage1_b,
           bn1_g, bn1_b, sage2_Wl, sage2_Wr, sage2_b, bn2_g, bn2_b,
           sage3_Wl, sage3_Wr, sage3_b, bn3_g, bn3_b,
           svi_W1, svi_b1, svi_W2, svi_b2):
    f32 = jnp.float32
    r1c = lambda a: a.reshape(1, -1).astype(f32)

    ctx = context_features
    w1t = ctx_W1.T
    src = edge_index[0]
    dst = edge_index[1]
    zeros32 = jnp.zeros((_RPT, 32), f32)
    zeros16 = jnp.zeros((_RPT, 16), f32)
    ones16 = jnp.ones((1000, 16), f32)

    p1a, p1b, r1 = pl.pallas_call(
        _pre_body,
        grid=(_GRID,),
        in_specs=[_rows(128), _rows(5), _full((5, 32)), _full((1, 32)),
                  _full((32, 32)), _full((1, 32)), _full((32, 128)), _full((1, 128)),
                  _full((1, 128)), _full((1, 128)), _full((1, 128)),
                  _full((128, 64)), _full((1, 64)), _full((64, 64)), _full((1, 64)),
                  _full((64, 64)), _full((64, 64)), _full((1, 64))],
        out_specs=[_rows(32), _rows(32), _rows(64)],
        out_shape=[jax.ShapeDtypeStruct((_N, 32), f32),
                   jax.ShapeDtypeStruct((_N, 32), f32),
                   jax.ShapeDtypeStruct((_N, 64), f32)],
    )(accessibility_features, ctx, w1t, r1c(ctx_b1), ctx_W2.T, r1c(ctx_b2),
      att_W.T, r1c(att_b), r1c(base_importance), r1c(ln_g), r1c(ln_b),
      enc_W1.T, r1c(enc_b1), enc_W2.T, r1c(enc_b2),
      sage1_Wl.T, sage1_Wr.T, r1c(sage1_b))

    cnt2 = _make_cnt(1000)(dst, ones16, zeros16)

    agg64 = _make_agg(32, 400, 5)
    s1 = agg64(p1a, p1b, src, dst, zeros32)

    p2a, p2b, r2, inv = pl.pallas_call(
        _mid1_body,
        grid=(_GRID,),
        in_specs=[_rows(128), _rows(64), _rows(128),
                  _full((1, 64)), _full((1, 64)),
                  _full((64, 64)), _full((64, 64)), _full((1, 64))],
        out_specs=[_rows(32), _rows(32), _rows(64), _rows(1)],
        out_shape=[jax.ShapeDtypeStruct((_N, 32), f32),
                   jax.ShapeDtypeStruct((_N, 32), f32),
                   jax.ShapeDtypeStruct((_N, 64), f32),
                   jax.ShapeDtypeStruct((_N, 1), f32)],
    )(s1, r1, cnt2, r1c(bn1_g), r1c(bn1_b),
      sage2_Wl.T, sage2_Wr.T, r1c(sage2_b))

    s2 = agg64(p2a, p2b, src, dst, zeros32)

    p3a, p3b, r3 = pl.pallas_call(
        _mid2_body,
        grid=(_GRID,),
        in_specs=[_rows(128), _rows(64), _rows(1),
                  _full((1, 64)), _full((1, 64)),
                  _full((64, 32)), _full((64, 32)), _full((1, 32))],
        out_specs=[_rows(16), _rows(16), _rows(32)],
        out_shape=[jax.ShapeDtypeStruct((_N, 16), f32),
                   jax.ShapeDtypeStruct((_N, 16), f32),
                   jax.ShapeDtypeStruct((_N, 32), f32)],
    )(s2, r2, inv, r1c(bn2_g), r1c(bn2_b),
      sage3_Wl.T, sage3_Wr.T, r1c(sage3_b))

    s3 = _make_agg(16, 1000, 10)(p3a, p3b, src, dst, zeros16)

    svi = pl.pallas_call(
        _final_body,
        grid=(_GRID,),
        in_specs=[_rows(128), _rows(32), _rows(1),
                  _full((1, 32)), _full((1, 32)),
                  _full((32, 16)), _full((1, 16)), _full((16, 1)), _full((1, 1))],
        out_specs=[_rows(1)],
        out_shape=[jax.ShapeDtypeStruct((_N, 1), f32)],
    )(s3, r3, inv, r1c(bn3_g), r1c(bn3_b),
      svi_W1.T, r1c(svi_b1), svi_W2.T, r1c(svi_b2))[0]

    return svi[:, 0]


# speedup vs baseline: 1.3049x; 1.1252x over previous
import functools
import math

import jax
import jax.numpy as jnp
from jax import lax
from jax.experimental import pallas as pl
from jax.experimental.pallas import tpu as pltpu
from jax.experimental.pallas import tpu_sc as plsc

_BN_SCALE = 1.0 / math.sqrt(1.0 + 1e-5)

_N = 50000
_E = 800000
_BLK = 2000
_GRID = _N // _BLK
_NT = 16
_NS = 50048
_RPT = _NS // _NT


def _full(spec_shape):
    nd = len(spec_shape)
    return pl.BlockSpec(spec_shape, lambda i, _n=nd: (0,) * _n)


def _rows(w):
    return pl.BlockSpec((_BLK, w), lambda i: (i, 0))


def _inv_from(cc):
    cnt = cc[:, 0:1] + cc[:, 16:17]
    return 1.0 / jnp.maximum(cnt, 1.0)


def _pre_body(feat, ctx, w1t, b1, w2t, b2, awt, ab, imp, lng, lnb,
              e1t, eb1, e2t, eb2, wlt, wrt, sb, out_ref):
    ce = jax.nn.relu(jnp.dot(ctx[...], w1t[...], preferred_element_type=jnp.float32) + b1[...])
    ce = jnp.dot(ce, w2t[...], preferred_element_type=jnp.float32) + b2[...]
    logits = jnp.dot(ce, awt[...], preferred_element_type=jnp.float32) + ab[...]
    att = jax.nn.softmax(logits, axis=-1)
    x = feat[...] * (att * imp[...])
    m = jnp.mean(x, axis=-1, keepdims=True)
    v = jnp.mean((x - m) * (x - m), axis=-1, keepdims=True)
    x = (x - m) / jnp.sqrt(v + 1e-5) * lng[...] + lnb[...]
    x = jax.nn.relu(jnp.dot(x, e1t[...], preferred_element_type=jnp.float32) + eb1[...])
    x = jax.nn.relu(jnp.dot(x, e2t[...], preferred_element_type=jnp.float32) + eb2[...])
    pp = jnp.dot(x, wlt[...], preferred_element_type=jnp.float32)
    rr = jnp.dot(x, wrt[...], preferred_element_type=jnp.float32) + sb[...]
    out_ref[...] = jnp.concatenate([pp, rr], axis=1)


def _mid_body(s_, prev, c01, bng, bnb, wlt, wrt, b_, out_ref):
    inv = _inv_from(c01[...])
    x = jax.nn.relu((s_[...][:, :64] * inv + prev[...][:, 64:128])
                    * (bng[...] * _BN_SCALE) + bnb[...])
    pp = jnp.dot(x, wlt[...], preferred_element_type=jnp.float32)
    rr = jnp.dot(x, wrt[...], preferred_element_type=jnp.float32) + b_[...]
    if pp.shape[1] + rr.shape[1] < 128:
        pad = jnp.zeros((pp.shape[0], 128 - pp.shape[1] - rr.shape[1]), jnp.float32)
        out_ref[...] = jnp.concatenate([pp, rr, pad], axis=1)
    else:
        out_ref[...] = jnp.concatenate([pp, rr], axis=1)


def _final_body(s_, prev, c01, bng, bnb, w1t, b1, w2t, b2, out_ref):
    inv = _inv_from(c01[...])
    x = jax.nn.relu((s_[...][:, :32] * inv + prev[...][:, 32:64])
                    * (bng[...] * _BN_SCALE) + bnb[...])
    h = jax.nn.relu(jnp.dot(x, w1t[...], preferred_element_type=jnp.float32) + b1[...])
    z = jnp.dot(h, w2t[...], preferred_element_type=jnp.float32) + b2[...]
    out_ref[...] = jnp.broadcast_to(jax.nn.sigmoid(z), (z.shape[0], 8))


def _make_agg(wh, ch, m_per_slab):
    mesh = plsc.VectorSubcoreMesh(core_axis_name="c", subcore_axis_name="s")
    ept = _E // _NT
    n_chunks = ept // ch
    n_slab = n_chunks // m_per_slab
    M = m_per_slab

    @functools.partial(
        pl.kernel,
        out_type=jax.ShapeDtypeStruct((_NS, 128), jnp.float32),
        mesh=mesh,
        scratch_types=[
            pltpu.VMEM((M * ch,), jnp.int32),
            pltpu.VMEM((M * ch,), jnp.int32),
            pltpu.VMEM((ch, wh), jnp.float32),
            pltpu.VMEM((ch, wh), jnp.float32),
            pltpu.VMEM_SHARED((_NS, wh), jnp.float32),
            pltpu.SemaphoreType.DMA,
            pltpu.SemaphoreType.DMA,
            pltpu.SemaphoreType.DMA,
            pltpu.SemaphoreType.DMA,
        ],
        compiler_params=pltpu.CompilerParams(use_tc_tiling_on_sc=False),
    )
    def agg(p_hbm, srcA, srcB, dst, zeros_hbm, out,
            src2, dst2, rows0, rows1, acc, g0, g1, s0, s1):
        c = lax.axis_index("c")
        s = lax.axis_index("s")
        pltpu.sync_copy(zeros_hbm, acc.at[pl.ds(s * _RPT, _RPT)])
        plsc.subcore_barrier()
        r0 = s * n_chunks
        rbuf = (rows0, rows1)
        gsem = (g0, g1)
        ssem = (s0, s1)

        def slab(t, carry):
            e = (r0 + t * M) * ch

            @pl.when(c == 0)
            def _():
                pltpu.sync_copy(srcA.at[pl.ds(e, M * ch)], src2)

            @pl.when(c == 1)
            def _():
                pltpu.sync_copy(srcB.at[pl.ds(e, M * ch)], src2)

            pltpu.sync_copy(dst.at[pl.ds(e, M * ch)], dst2)
            h_g = [None] * M
            h_s = [None] * M
            h_g[0] = pltpu.async_copy(
                p_hbm.at[src2.at[pl.ds(0, ch)]], rbuf[0], gsem[0])
            for m in range(M):
                h_g[m].wait()
                if m + 1 < M:
                    if m >= 1:
                        h_s[m - 1].wait()
                    h_g[m + 1] = pltpu.async_copy(
                        p_hbm.at[src2.at[pl.ds((m + 1) * ch, ch)]],
                        rbuf[(m + 1) % 2], gsem[(m + 1) % 2])
                h_s[m] = pltpu.async_copy(
                    rbuf[m % 2], acc.at[dst2.at[pl.ds(m * ch, ch)]],
                    ssem[m % 2], add=True)
            if M >= 2:
                h_s[M - 2].wait()
            h_s[M - 1].wait()
            return carry

        lax.fori_loop(0, n_slab, slab, 0)
        plsc.subcore_barrier()

        @pl.when(c == 0)
        def _():
            pltpu.sync_copy(acc.at[pl.ds(s * _RPT, _RPT)],
                            out.at[pl.ds(s * _RPT, _RPT), pl.ds(0, wh)])

        @pl.when(c == 1)
        def _():
            pltpu.sync_copy(acc.at[pl.ds(s * _RPT, _RPT)],
                            out.at[pl.ds(s * _RPT, _RPT), pl.ds(wh, wh)])

    return agg


def _make_cnt(ch):
    mesh = plsc.VectorSubcoreMesh(core_axis_name="c", subcore_axis_name="s")
    ept = _E // 2 // _NT
    n_chunks = ept // ch

    @functools.partial(
        pl.kernel,
        out_type=jax.ShapeDtypeStruct((_NS, 128), jnp.float32),
        mesh=mesh,
        scratch_types=[
            pltpu.VMEM((n_chunks * ch,), jnp.int32),
            pltpu.VMEM((ch, 16), jnp.float32),
            pltpu.VMEM_SHARED((_NS, 16), jnp.float32),
            pltpu.SemaphoreType.DMA,
        ],
        compiler_params=pltpu.CompilerParams(use_tc_tiling_on_sc=False),
    )
    def cnt_k(dst, ones_hbm, zeros_hbm, out, dst2, ones_v, acc, sem):
        c = lax.axis_index("c")
        s = lax.axis_index("s")
        pltpu.sync_copy(ones_hbm, ones_v)
        pltpu.sync_copy(zeros_hbm, acc.at[pl.ds(s * _RPT, _RPT)])
        plsc.subcore_barrier()
        e0 = c * (_E // 2) + s * ept
        pltpu.sync_copy(dst.at[pl.ds(e0, n_chunks * ch)], dst2)
        hs = [pltpu.async_copy(ones_v, acc.at[dst2.at[pl.ds(m * ch, ch)]],
                               sem, add=True)
              for m in range(n_chunks)]
        for h in hs:
            h.wait()
        plsc.subcore_barrier()

        @pl.when(c == 0)
        def _():
            pltpu.sync_copy(acc.at[pl.ds(s * _RPT, _RPT)],
                            out.at[pl.ds(s * _RPT, _RPT), pl.ds(0, 16)])

        @pl.when(c == 1)
        def _():
            pltpu.sync_copy(acc.at[pl.ds(s * _RPT, _RPT)],
                            out.at[pl.ds(s * _RPT, _RPT), pl.ds(16, 16)])

    return cnt_k


def kernel(accessibility_features, edge_index, context_features, ctx_W1, ctx_b1,
           ctx_W2, ctx_b2, att_W, att_b, base_importance, ln_g, ln_b,
           enc_W1, enc_b1, enc_W2, enc_b2, sage1_Wl, sage1_Wr, sage1_b,
           bn1_g, bn1_b, sage2_Wl, sage2_Wr, sage2_b, bn2_g, bn2_b,
           sage3_Wl, sage3_Wr, sage3_b, bn3_g, bn3_b,
           svi_W1, svi_b1, svi_W2, svi_b2):
    f32 = jnp.float32
    r1c = lambda a: a.reshape(1, -1).astype(f32)

    src = edge_index[0]
    dst = edge_index[1]
    srcA4 = src * 4
    srcB4 = srcA4 + 1
    srcA8 = src * 8
    srcB8 = srcA8 + 1
    zeros32 = jnp.zeros((_RPT, 32), f32)
    zeros16 = jnp.zeros((_RPT, 16), f32)
    ones16 = jnp.ones((1000, 16), f32)

    cnt2 = _make_cnt(1000)(dst, ones16, zeros16)
    cnt_spec = _rows(128)

    P1 = pl.pallas_call(
        _pre_body,
        grid=(_GRID,),
        in_specs=[_rows(128), _rows(5), _full((5, 32)), _full((1, 32)),
                  _full((32, 32)), _full((1, 32)), _full((32, 128)), _full((1, 128)),
                  _full((1, 128)), _full((1, 128)), _full((1, 128)),
                  _full((128, 64)), _full((1, 64)), _full((64, 64)), _full((1, 64)),
                  _full((64, 64)), _full((64, 64)), _full((1, 64))],
        out_specs=[_rows(128)],
        out_shape=[jax.ShapeDtypeStruct((_N, 128), f32)],
    )(accessibility_features, context_features, ctx_W1.T, r1c(ctx_b1),
      ctx_W2.T, r1c(ctx_b2), att_W.T, r1c(att_b), r1c(base_importance),
      r1c(ln_g), r1c(ln_b), enc_W1.T, r1c(enc_b1), enc_W2.T, r1c(enc_b2),
      sage1_Wl.T, sage1_Wr.T, r1c(sage1_b))[0]

    agg64 = _make_agg(32, 400, 5)
    s1 = agg64(P1.reshape(4 * _N, 32), srcA4, srcB4, dst, zeros32)

    P2 = pl.pallas_call(
        _mid_body,
        grid=(_GRID,),
        in_specs=[_rows(128), _rows(128), cnt_spec,
                  _full((1, 64)), _full((1, 64)),
                  _full((64, 64)), _full((64, 64)), _full((1, 64))],
        out_specs=[_rows(128)],
        out_shape=[jax.ShapeDtypeStruct((_N, 128), f32)],
    )(s1, P1, cnt2, r1c(bn1_g), r1c(bn1_b),
      sage2_Wl.T, sage2_Wr.T, r1c(sage2_b))[0]

    s2 = agg64(P2.reshape(4 * _N, 32), srcA4, srcB4, dst, zeros32)

    P3 = pl.pallas_call(
        _mid_body,
        grid=(_GRID,),
        in_specs=[_rows(128), _rows(128), cnt_spec,
                  _full((1, 64)), _full((1, 64)),
                  _full((64, 32)), _full((64, 32)), _full((1, 32))],
        out_specs=[_rows(128)],
        out_shape=[jax.ShapeDtypeStruct((_N, 128), f32)],
    )(s2, P2, cnt2, r1c(bn2_g), r1c(bn2_b),
      sage3_Wl.T, sage3_Wr.T, r1c(sage3_b))[0]

    s3 = _make_agg(16, 1000, 10)(P3.reshape(8 * _N, 16), srcA8, srcB8, dst, zeros16)

    svi = pl.pallas_call(
        _final_body,
        grid=(_GRID,),
        in_specs=[_rows(128), _rows(128), cnt_spec,
                  _full((1, 32)), _full((1, 32)),
                  _full((32, 16)), _full((1, 16)), _full((16, 1)), _full((1, 1))],
        out_specs=[_rows(8)],
        out_shape=[jax.ShapeDtypeStruct((_N, 8), f32)],
    )(s3, P3, cnt2, r1c(bn3_g), r1c(bn3_b),
      svi_W1.T, r1c(svi_b1), svi_W2.T, r1c(svi_b2))[0]

    return svi[:, 0]


# speedup vs baseline: 1.3463x; 1.0317x over previous
import functools
import math

import jax
import jax.numpy as jnp
from jax import lax
from jax.experimental import pallas as pl
from jax.experimental.pallas import tpu as pltpu
from jax.experimental.pallas import tpu_sc as plsc

_BN_SCALE = 1.0 / math.sqrt(1.0 + 1e-5)

_N = 50000
_E = 800000
_BLK = 10000
_GRID = _N // _BLK
_NT = 16
_NS = 50048
_RPT = _NS // _NT


def _full(spec_shape):
    nd = len(spec_shape)
    return pl.BlockSpec(spec_shape, lambda i, _n=nd: (0,) * _n)


def _rows(w):
    return pl.BlockSpec((_BLK, w), lambda i: (i, 0))


def _inv_from(cc):
    cnt = cc[:, 0:1] + cc[:, 16:17]
    return 1.0 / jnp.maximum(cnt, 1.0)


def _pre_body(feat, ctx, w1t, b1, w2t, b2, awt, ab, imp, lng, lnb,
              e1t, eb1, e2t, eb2, wlt, wrt, sb, out_ref):
    ce = jax.nn.relu(jnp.dot(ctx[...], w1t[...], preferred_element_type=jnp.float32) + b1[...])
    ce = jnp.dot(ce, w2t[...], preferred_element_type=jnp.float32) + b2[...]
    logits = jnp.dot(ce, awt[...], preferred_element_type=jnp.float32) + ab[...]
    att = jax.nn.softmax(logits, axis=-1)
    x = feat[...] * (att * imp[...])
    m = jnp.mean(x, axis=-1, keepdims=True)
    v = jnp.mean((x - m) * (x - m), axis=-1, keepdims=True)
    x = (x - m) / jnp.sqrt(v + 1e-5) * lng[...] + lnb[...]
    x = jax.nn.relu(jnp.dot(x, e1t[...], preferred_element_type=jnp.float32) + eb1[...])
    x = jax.nn.relu(jnp.dot(x, e2t[...], preferred_element_type=jnp.float32) + eb2[...])
    pp = jnp.dot(x, wlt[...], preferred_element_type=jnp.float32)
    rr = jnp.dot(x, wrt[...], preferred_element_type=jnp.float32) + sb[...]
    out_ref[...] = jnp.concatenate([pp, rr], axis=1)


def _mid_body(s_, prev, c01, bng, bnb, wlt, wrt, b_, out_ref):
    inv = _inv_from(c01[...])
    x = jax.nn.relu((s_[...][:, :64] * inv + prev[...][:, 64:128])
                    * (bng[...] * _BN_SCALE) + bnb[...])
    pp = jnp.dot(x, wlt[...], preferred_element_type=jnp.float32)
    rr = jnp.dot(x, wrt[...], preferred_element_type=jnp.float32) + b_[...]
    if pp.shape[1] + rr.shape[1] < 128:
        pad = jnp.zeros((pp.shape[0], 128 - pp.shape[1] - rr.shape[1]), jnp.float32)
        out_ref[...] = jnp.concatenate([pp, rr, pad], axis=1)
    else:
        out_ref[...] = jnp.concatenate([pp, rr], axis=1)


def _final_body(s_, prev, c01, bng, bnb, w1t, b1, w2t, b2, out_ref):
    inv = _inv_from(c01[...])
    x = jax.nn.relu((s_[...][:, :32] * inv + prev[...][:, 32:64])
                    * (bng[...] * _BN_SCALE) + bnb[...])
    h = jax.nn.relu(jnp.dot(x, w1t[...], preferred_element_type=jnp.float32) + b1[...])
    z = jnp.dot(h, w2t[...], preferred_element_type=jnp.float32) + b2[...]
    out_ref[...] = jnp.broadcast_to(jax.nn.sigmoid(z), (z.shape[0], 8))


def _make_agg(wh, ch, m_per_slab):
    mesh = plsc.VectorSubcoreMesh(core_axis_name="c", subcore_axis_name="s")
    ept = _E // _NT
    n_chunks = ept // ch
    n_slab = n_chunks // m_per_slab
    M = m_per_slab

    @functools.partial(
        pl.kernel,
        out_type=jax.ShapeDtypeStruct((_NS, 128), jnp.float32),
        mesh=mesh,
        scratch_types=[
            pltpu.VMEM((M * ch,), jnp.int32),
            pltpu.VMEM((M * ch,), jnp.int32),
            pltpu.VMEM((ch, wh), jnp.float32),
            pltpu.VMEM((ch, wh), jnp.float32),
            pltpu.VMEM_SHARED((_NS, wh), jnp.float32),
            pltpu.SemaphoreType.DMA,
            pltpu.SemaphoreType.DMA,
            pltpu.SemaphoreType.DMA,
            pltpu.SemaphoreType.DMA,
        ],
        compiler_params=pltpu.CompilerParams(use_tc_tiling_on_sc=False),
    )
    def agg(p_hbm, srcA, srcB, dst, zeros_hbm, out,
            src2, dst2, rows0, rows1, acc, g0, g1, s0, s1):
        c = lax.axis_index("c")
        s = lax.axis_index("s")
        pltpu.sync_copy(zeros_hbm, acc.at[pl.ds(s * _RPT, _RPT)])
        plsc.subcore_barrier()
        r0 = s * n_chunks
        rbuf = (rows0, rows1)
        gsem = (g0, g1)
        ssem = (s0, s1)

        def slab(t, carry):
            e = (r0 + t * M) * ch

            @pl.when(c == 0)
            def _():
                pltpu.sync_copy(srcA.at[pl.ds(e, M * ch)], src2)

            @pl.when(c == 1)
            def _():
                pltpu.sync_copy(srcB.at[pl.ds(e, M * ch)], src2)

            pltpu.sync_copy(dst.at[pl.ds(e, M * ch)], dst2)
            h_g = [None] * M
            h_s = [None] * M
            h_g[0] = pltpu.async_copy(
                p_hbm.at[src2.at[pl.ds(0, ch)]], rbuf[0], gsem[0])
            for m in range(M):
                h_g[m].wait()
                if m + 1 < M:
                    if m >= 1:
                        h_s[m - 1].wait()
                    h_g[m + 1] = pltpu.async_copy(
                        p_hbm.at[src2.at[pl.ds((m + 1) * ch, ch)]],
                        rbuf[(m + 1) % 2], gsem[(m + 1) % 2])
                h_s[m] = pltpu.async_copy(
                    rbuf[m % 2], acc.at[dst2.at[pl.ds(m * ch, ch)]],
                    ssem[m % 2], add=True)
            if M >= 2:
                h_s[M - 2].wait()
            h_s[M - 1].wait()
            return carry

        lax.fori_loop(0, n_slab, slab, 0)
        plsc.subcore_barrier()

        @pl.when(c == 0)
        def _():
            pltpu.sync_copy(acc.at[pl.ds(s * _RPT, _RPT)],
                            out.at[pl.ds(s * _RPT, _RPT), pl.ds(0, wh)])

        @pl.when(c == 1)
        def _():
            pltpu.sync_copy(acc.at[pl.ds(s * _RPT, _RPT)],
                            out.at[pl.ds(s * _RPT, _RPT), pl.ds(wh, wh)])

    return agg


def _make_cnt(ch):
    mesh = plsc.VectorSubcoreMesh(core_axis_name="c", subcore_axis_name="s")
    ept = _E // 2 // _NT
    n_chunks = ept // ch

    @functools.partial(
        pl.kernel,
        out_type=jax.ShapeDtypeStruct((_NS, 128), jnp.float32),
        mesh=mesh,
        scratch_types=[
            pltpu.VMEM((n_chunks * ch,), jnp.int32),
            pltpu.VMEM((ch, 16), jnp.float32),
            pltpu.VMEM_SHARED((_NS, 16), jnp.float32),
            pltpu.SemaphoreType.DMA,
        ],
        compiler_params=pltpu.CompilerParams(use_tc_tiling_on_sc=False),
    )
    def cnt_k(dst, ones_hbm, zeros_hbm, out, dst2, ones_v, acc, sem):
        c = lax.axis_index("c")
        s = lax.axis_index("s")
        pltpu.sync_copy(ones_hbm, ones_v)
        pltpu.sync_copy(zeros_hbm, acc.at[pl.ds(s * _RPT, _RPT)])
        plsc.subcore_barrier()
        e0 = c * (_E // 2) + s * ept
        pltpu.sync_copy(dst.at[pl.ds(e0, n_chunks * ch)], dst2)
        hs = [pltpu.async_copy(ones_v, acc.at[dst2.at[pl.ds(m * ch, ch)]],
                               sem, add=True)
              for m in range(n_chunks)]
        for h in hs:
            h.wait()
        plsc.subcore_barrier()

        @pl.when(c == 0)
        def _():
            pltpu.sync_copy(acc.at[pl.ds(s * _RPT, _RPT)],
                            out.at[pl.ds(s * _RPT, _RPT), pl.ds(0, 16)])

        @pl.when(c == 1)
        def _():
            pltpu.sync_copy(acc.at[pl.ds(s * _RPT, _RPT)],
                            out.at[pl.ds(s * _RPT, _RPT), pl.ds(16, 16)])

    return cnt_k


def kernel(accessibility_features, edge_index, context_features, ctx_W1, ctx_b1,
           ctx_W2, ctx_b2, att_W, att_b, base_importance, ln_g, ln_b,
           enc_W1, enc_b1, enc_W2, enc_b2, sage1_Wl, sage1_Wr, sage1_b,
           bn1_g, bn1_b, sage2_Wl, sage2_Wr, sage2_b, bn2_g, bn2_b,
           sage3_Wl, sage3_Wr, sage3_b, bn3_g, bn3_b,
           svi_W1, svi_b1, svi_W2, svi_b2):
    f32 = jnp.float32
    r1c = lambda a: a.reshape(1, -1).astype(f32)

    src = edge_index[0]
    dst = edge_index[1]
    srcA4 = src * 4
    srcB4 = srcA4 + 1
    srcA8 = src * 8
    srcB8 = srcA8 + 1
    zeros32 = jnp.zeros((_RPT, 32), f32)
    zeros16 = jnp.zeros((_RPT, 16), f32)
    ones16 = jnp.ones((1000, 16), f32)

    cnt2 = _make_cnt(1000)(dst, ones16, zeros16)
    cnt_spec = _rows(128)

    P1 = pl.pallas_call(
        _pre_body,
        grid=(_GRID,),
        in_specs=[_rows(128), _rows(5), _full((5, 32)), _full((1, 32)),
                  _full((32, 32)), _full((1, 32)), _full((32, 128)), _full((1, 128)),
                  _full((1, 128)), _full((1, 128)), _full((1, 128)),
                  _full((128, 64)), _full((1, 64)), _full((64, 64)), _full((1, 64)),
                  _full((64, 64)), _full((64, 64)), _full((1, 64))],
        out_specs=[_rows(128)],
        out_shape=[jax.ShapeDtypeStruct((_N, 128), f32)],
    )(accessibility_features, context_features, ctx_W1.T, r1c(ctx_b1),
      ctx_W2.T, r1c(ctx_b2), att_W.T, r1c(att_b), r1c(base_importance),
      r1c(ln_g), r1c(ln_b), enc_W1.T, r1c(enc_b1), enc_W2.T, r1c(enc_b2),
      sage1_Wl.T, sage1_Wr.T, r1c(sage1_b))[0]

    agg64 = _make_agg(32, 400, 5)
    s1 = agg64(P1.reshape(4 * _N, 32), srcA4, srcB4, dst, zeros32)

    P2 = pl.pallas_call(
        _mid_body,
        grid=(_GRID,),
        in_specs=[_rows(128), _rows(128), cnt_spec,
                  _full((1, 64)), _full((1, 64)),
                  _full((64, 64)), _full((64, 64)), _full((1, 64))],
        out_specs=[_rows(128)],
        out_shape=[jax.ShapeDtypeStruct((_N, 128), f32)],
    )(s1, P1, cnt2, r1c(bn1_g), r1c(bn1_b),
      sage2_Wl.T, sage2_Wr.T, r1c(sage2_b))[0]

    s2 = agg64(P2.reshape(4 * _N, 32), srcA4, srcB4, dst, zeros32)

    P3 = pl.pallas_call(
        _mid_body,
        grid=(_GRID,),
        in_specs=[_rows(128), _rows(128), cnt_spec,
                  _full((1, 64)), _full((1, 64)),
                  _full((64, 32)), _full((64, 32)), _full((1, 32))],
        out_specs=[_rows(128)],
        out_shape=[jax.ShapeDtypeStruct((_N, 128), f32)],
    )(s2, P2, cnt2, r1c(bn2_g), r1c(bn2_b),
      sage3_Wl.T, sage3_Wr.T, r1c(sage3_b))[0]

    s3 = _make_agg(16, 1000, 10)(P3.reshape(8 * _N, 16), srcA8, srcB8, dst, zeros16)

    svi = pl.pallas_call(
        _final_body,
        grid=(_GRID,),
        in_specs=[_rows(128), _rows(128), cnt_spec,
                  _full((1, 32)), _full((1, 32)),
                  _full((32, 16)), _full((1, 16)), _full((16, 1)), _full((1, 1))],
        out_specs=[_rows(8)],
        out_shape=[jax.ShapeDtypeStruct((_N, 8), f32)],
    )(s3, P3, cnt2, r1c(bn3_g), r1c(bn3_b),
      svi_W1.T, r1c(svi_b1), svi_W2.T, r1c(svi_b2))[0]

    return svi[:, 0]


# speedup vs baseline: 1.3645x; 1.0135x over previous
import functools
import math

import jax
import jax.numpy as jnp
from jax import lax
from jax.experimental import pallas as pl
from jax.experimental.pallas import tpu as pltpu
from jax.experimental.pallas import tpu_sc as plsc

_BN_SCALE = 1.0 / math.sqrt(1.0 + 1e-5)

_N = 50000
_E = 800000
_BLK = 10000
_GRID = _N // _BLK
_NT = 16
_NS = 50048
_RPT = _NS // _NT


def _full(spec_shape):
    nd = len(spec_shape)
    return pl.BlockSpec(spec_shape, lambda i, _n=nd: (0,) * _n)


def _rows(w):
    return pl.BlockSpec((_BLK, w), lambda i: (i, 0))


def _inv_from(cc):
    cnt = cc[:, 0:1] + cc[:, 16:17]
    return 1.0 / jnp.maximum(cnt, 1.0)


def _pre_body(feat, ctx, w1t, b1, w2t, b2, awt, ab, imp, lng, lnb,
              e1t, eb1, e2t, eb2, wlt, wrt, sb, out_ref):
    ce = jax.nn.relu(jnp.dot(ctx[...], w1t[...], preferred_element_type=jnp.float32) + b1[...])
    ce = jnp.dot(ce, w2t[...], preferred_element_type=jnp.float32) + b2[...]
    logits = jnp.dot(ce, awt[...], preferred_element_type=jnp.float32) + ab[...]
    att = jax.nn.softmax(logits, axis=-1)
    x = feat[...] * (att * imp[...])
    m = jnp.mean(x, axis=-1, keepdims=True)
    v = jnp.mean((x - m) * (x - m), axis=-1, keepdims=True)
    x = (x - m) / jnp.sqrt(v + 1e-5) * lng[...] + lnb[...]
    x = jax.nn.relu(jnp.dot(x, e1t[...], preferred_element_type=jnp.float32) + eb1[...])
    x = jax.nn.relu(jnp.dot(x, e2t[...], preferred_element_type=jnp.float32) + eb2[...])
    pp = jnp.dot(x, wlt[...], preferred_element_type=jnp.float32)
    rr = jnp.dot(x, wrt[...], preferred_element_type=jnp.float32) + sb[...]
    out_ref[...] = jnp.concatenate([pp, rr], axis=1)


def _mid_body(s_, prev, c01, bng, bnb, wlt, wrt, b_, out_ref):
    inv = _inv_from(c01[...])
    x = jax.nn.relu((s_[...][:, :64] * inv + prev[...][:, 64:128])
                    * (bng[...] * _BN_SCALE) + bnb[...])
    pp = jnp.dot(x, wlt[...], preferred_element_type=jnp.float32)
    rr = jnp.dot(x, wrt[...], preferred_element_type=jnp.float32) + b_[...]
    used = pp.shape[1] + rr.shape[1]
    if used < 128:
        pad = jnp.zeros((pp.shape[0], 128 - used - 1), jnp.float32)
        out_ref[...] = jnp.concatenate([pp, rr, inv, pad], axis=1)
    else:
        out_ref[...] = jnp.concatenate([pp, rr], axis=1)


def _final_body(s_, prev, bng, bnb, w1t, b1, w2t, b2, out_ref):
    pv = prev[...]
    inv = pv[:, 64:65]
    x = jax.nn.relu((s_[...][:, :32] * inv + pv[:, 32:64])
                    * (bng[...] * _BN_SCALE) + bnb[...])
    h = jax.nn.relu(jnp.dot(x, w1t[...], preferred_element_type=jnp.float32) + b1[...])
    z = jnp.dot(h, w2t[...], preferred_element_type=jnp.float32) + b2[...]
    out_ref[...] = jnp.broadcast_to(jax.nn.sigmoid(z), (z.shape[0], 8))


def _make_agg(wh, ch, m_per_slab):
    mesh = plsc.VectorSubcoreMesh(core_axis_name="c", subcore_axis_name="s")
    ept = _E // _NT
    n_chunks = ept // ch
    n_slab = n_chunks // m_per_slab
    M = m_per_slab
    n_zfull = _RPT // ch
    z_rem = _RPT % ch

    @functools.partial(
        pl.kernel,
        out_type=jax.ShapeDtypeStruct((_NS, 128), jnp.float32),
        mesh=mesh,
        scratch_types=[
            pltpu.VMEM((M * ch,), jnp.int32),
            pltpu.VMEM((M * ch,), jnp.int32),
            pltpu.VMEM((ch, wh), jnp.float32),
            pltpu.VMEM((ch, wh), jnp.float32),
            pltpu.VMEM_SHARED((_NS, wh), jnp.float32),
            pltpu.SemaphoreType.DMA,
            pltpu.SemaphoreType.DMA,
            pltpu.SemaphoreType.DMA,
            pltpu.SemaphoreType.DMA,
        ],
        compiler_params=pltpu.CompilerParams(use_tc_tiling_on_sc=False),
    )
    def agg(p_hbm, srcA, srcB, dst, out,
            src2, dst2, rows0, rows1, acc, g0, g1, s0, s1):
        c = lax.axis_index("c")
        s = lax.axis_index("s")
        z16 = jnp.zeros((16,), jnp.float32)

        def zrow(i, carry):
            for j in range(wh // 16):
                rows0[i, pl.ds(j * 16, 16)] = z16
            return carry

        lax.fori_loop(0, ch, zrow, 0)
        base = s * _RPT
        for k in range(n_zfull):
            pltpu.sync_copy(rows0, acc.at[pl.ds(base + k * ch, ch)])
        if z_rem:
            pltpu.sync_copy(rows0.at[pl.ds(0, z_rem)],
                            acc.at[pl.ds(base + n_zfull * ch, z_rem)])
        plsc.subcore_barrier()
        r0 = s * n_chunks
        rbuf = (rows0, rows1)
        gsem = (g0, g1)
        ssem = (s0, s1)

        def slab(t, carry):
            e = (r0 + t * M) * ch

            @pl.when(c == 0)
            def _():
                pltpu.sync_copy(srcA.at[pl.ds(e, M * ch)], src2)

            @pl.when(c == 1)
            def _():
                pltpu.sync_copy(srcB.at[pl.ds(e, M * ch)], src2)

            pltpu.sync_copy(dst.at[pl.ds(e, M * ch)], dst2)
            h_g = [None] * M
            h_s = [None] * M
            h_g[0] = pltpu.async_copy(
                p_hbm.at[src2.at[pl.ds(0, ch)]], rbuf[0], gsem[0])
            for m in range(M):
                h_g[m].wait()
                if m + 1 < M:
                    if m >= 1:
                        h_s[m - 1].wait()
                    h_g[m + 1] = pltpu.async_copy(
                        p_hbm.at[src2.at[pl.ds((m + 1) * ch, ch)]],
                        rbuf[(m + 1) % 2], gsem[(m + 1) % 2])
                h_s[m] = pltpu.async_copy(
                    rbuf[m % 2], acc.at[dst2.at[pl.ds(m * ch, ch)]],
                    ssem[m % 2], add=True)
            if M >= 2:
                h_s[M - 2].wait()
            h_s[M - 1].wait()
            return carry

        lax.fori_loop(0, n_slab, slab, 0)
        plsc.subcore_barrier()

        @pl.when(c == 0)
        def _():
            pltpu.sync_copy(acc.at[pl.ds(s * _RPT, _RPT)],
                            out.at[pl.ds(s * _RPT, _RPT), pl.ds(0, wh)])

        @pl.when(c == 1)
        def _():
            pltpu.sync_copy(acc.at[pl.ds(s * _RPT, _RPT)],
                            out.at[pl.ds(s * _RPT, _RPT), pl.ds(wh, wh)])

    return agg


def _make_cnt(ch):
    mesh = plsc.VectorSubcoreMesh(core_axis_name="c", subcore_axis_name="s")
    ept = _E // 2 // _NT
    n_chunks = ept // ch

    @functools.partial(
        pl.kernel,
        out_type=jax.ShapeDtypeStruct((_NS, 128), jnp.float32),
        mesh=mesh,
        scratch_types=[
            pltpu.VMEM((n_chunks * ch,), jnp.int32),
            pltpu.VMEM((ch, 16), jnp.float32),
            pltpu.VMEM((ch, 16), jnp.float32),
            pltpu.VMEM_SHARED((_NS, 16), jnp.float32),
            pltpu.SemaphoreType.DMA,
        ],
        compiler_params=pltpu.CompilerParams(use_tc_tiling_on_sc=False),
    )
    def cnt_k(dst, ones_hbm, out, dst2, ones_v, zv, acc, sem):
        c = lax.axis_index("c")
        s = lax.axis_index("s")
        z16 = jnp.zeros((16,), jnp.float32)

        def zrow(i, carry):
            zv[i, pl.ds(0, 16)] = z16
            return carry

        lax.fori_loop(0, ch, zrow, 0)
        pltpu.sync_copy(ones_hbm, ones_v)
        base = s * _RPT
        for k in range(_RPT // ch):
            pltpu.sync_copy(zv, acc.at[pl.ds(base + k * ch, ch)])
        if _RPT % ch:
            pltpu.sync_copy(zv.at[pl.ds(0, _RPT % ch)],
                            acc.at[pl.ds(base + (_RPT // ch) * ch, _RPT % ch)])
        plsc.subcore_barrier()
        e0 = c * (_E // 2) + s * ept
        pltpu.sync_copy(dst.at[pl.ds(e0, n_chunks * ch)], dst2)
        hs = [pltpu.async_copy(ones_v, acc.at[dst2.at[pl.ds(m * ch, ch)]],
                               sem, add=True)
              for m in range(n_chunks)]
        for h in hs:
            h.wait()
        plsc.subcore_barrier()

        @pl.when(c == 0)
        def _():
            pltpu.sync_copy(acc.at[pl.ds(s * _RPT, _RPT)],
                            out.at[pl.ds(s * _RPT, _RPT), pl.ds(0, 16)])

        @pl.when(c == 1)
        def _():
            pltpu.sync_copy(acc.at[pl.ds(s * _RPT, _RPT)],
                            out.at[pl.ds(s * _RPT, _RPT), pl.ds(16, 16)])

    return cnt_k


def kernel(accessibility_features, edge_index, context_features, ctx_W1, ctx_b1,
           ctx_W2, ctx_b2, att_W, att_b, base_importance, ln_g, ln_b,
           enc_W1, enc_b1, enc_W2, enc_b2, sage1_Wl, sage1_Wr, sage1_b,
           bn1_g, bn1_b, sage2_Wl, sage2_Wr, sage2_b, bn2_g, bn2_b,
           sage3_Wl, sage3_Wr, sage3_b, bn3_g, bn3_b,
           svi_W1, svi_b1, svi_W2, svi_b2):
    f32 = jnp.float32
    r1c = lambda a: a.reshape(1, -1).astype(f32)

    src = edge_index[0]
    dst = edge_index[1]
    srcA4 = src * 4
    srcB4 = srcA4 + 1
    srcA8 = src * 8
    srcB8 = srcA8 + 1
    ones16 = jnp.ones((1000, 16), f32)

    cnt2 = _make_cnt(1000)(dst, ones16)
    cnt_spec = _rows(128)

    P1 = pl.pallas_call(
        _pre_body,
        grid=(_GRID,),
        in_specs=[_rows(128), _rows(5), _full((5, 32)), _full((1, 32)),
                  _full((32, 32)), _full((1, 32)), _full((32, 128)), _full((1, 128)),
                  _full((1, 128)), _full((1, 128)), _full((1, 128)),
                  _full((128, 64)), _full((1, 64)), _full((64, 64)), _full((1, 64)),
                  _full((64, 64)), _full((64, 64)), _full((1, 64))],
        out_specs=[_rows(128)],
        out_shape=[jax.ShapeDtypeStruct((_N, 128), f32)],
    )(accessibility_features, context_features, ctx_W1.T, r1c(ctx_b1),
      ctx_W2.T, r1c(ctx_b2), att_W.T, r1c(att_b), r1c(base_importance),
      r1c(ln_g), r1c(ln_b), enc_W1.T, r1c(enc_b1), enc_W2.T, r1c(enc_b2),
      sage1_Wl.T, sage1_Wr.T, r1c(sage1_b))[0]

    agg64 = _make_agg(32, 400, 5)
    s1 = agg64(P1.reshape(4 * _N, 32), srcA4, srcB4, dst)

    P2 = pl.pallas_call(
        _mid_body,
        grid=(_GRID,),
        in_specs=[_rows(128), _rows(128), cnt_spec,
                  _full((1, 64)), _full((1, 64)),
                  _full((64, 64)), _full((64, 64)), _full((1, 64))],
        out_specs=[_rows(128)],
        out_shape=[jax.ShapeDtypeStruct((_N, 128), f32)],
    )(s1, P1, cnt2, r1c(bn1_g), r1c(bn1_b),
      sage2_Wl.T, sage2_Wr.T, r1c(sage2_b))[0]

    s2 = agg64(P2.reshape(4 * _N, 32), srcA4, srcB4, dst)

    P3 = pl.pallas_call(
        _mid_body,
        grid=(_GRID,),
        in_specs=[_rows(128), _rows(128), cnt_spec,
                  _full((1, 64)), _full((1, 64)),
                  _full((64, 32)), _full((64, 32)), _full((1, 32))],
        out_specs=[_rows(128)],
        out_shape=[jax.ShapeDtypeStruct((_N, 128), f32)],
    )(s2, P2, cnt2, r1c(bn2_g), r1c(bn2_b),
      sage3_Wl.T, sage3_Wr.T, r1c(sage3_b))[0]

    s3 = _make_agg(16, 1000, 10)(P3.reshape(8 * _N, 16), srcA8, srcB8, dst)

    svi = pl.pallas_call(
        _final_body,
        grid=(_GRID,),
        in_specs=[_rows(128), _rows(128),
                  _full((1, 32)), _full((1, 32)),
                  _full((32, 16)), _full((1, 16)), _full((16, 1)), _full((1, 1))],
        out_specs=[_rows(8)],
        out_shape=[jax.ShapeDtypeStruct((_N, 8), f32)],
    )(s3, P3, r1c(bn3_g), r1c(bn3_b),
      svi_W1.T, r1c(svi_b1), svi_W2.T, r1c(svi_b2))[0]

    return svi[:, 0]
